# baseline (speedup 1.0000x reference)
import functools

import jax
import jax.numpy as jnp
from jax import lax
from jax.experimental import pallas as pl
from jax.experimental.pallas import tpu as pltpu
from jax.experimental.pallas import tpu_sc as plsc

BATCH = 16384
FACTOR = 64
NW = 32
B_PER_W = BATCH // NW
N_CHUNK = 4
CHUNK = B_PER_W // N_CHUNK
GROUPS = B_PER_W // 16

_mesh = plsc.VectorSubcoreMesh(core_axis_name="c", subcore_axis_name="s")


@functools.partial(
    pl.kernel,
    mesh=_mesh,
    out_type=jax.ShapeDtypeStruct((BATCH,), jnp.float32),
    scratch_types=[
        pltpu.VMEM((N_CHUNK, CHUNK), jnp.int32),
        pltpu.VMEM((N_CHUNK, CHUNK), jnp.int32),
        pltpu.VMEM((B_PER_W, FACTOR), jnp.float32),
        pltpu.VMEM((B_PER_W, FACTOR), jnp.float32),
        pltpu.VMEM((B_PER_W,), jnp.float32),
        pltpu.SemaphoreType.DMA,
    ],
    compiler_params=pltpu.CompilerParams(
        needs_layout_passes=False, use_tc_tiling_on_sc=False
    ),
)
def _pointmf_sc(user_hbm, item_hbm, eu_hbm, ei_hbm, out_hbm,
                uidx, iidx, urows, irows, out_v, sem):
    wid = lax.axis_index("s") * 2 + lax.axis_index("c")
    base = wid * B_PER_W

    pltpu.sync_copy(user_hbm.at[wid], uidx)
    pltpu.sync_copy(item_hbm.at[wid], iidx)

    copies = []
    for j in range(N_CHUNK):
        dst = pl.ds(j * CHUNK, CHUNK)
        copies.append(pltpu.async_copy(eu_hbm.at[uidx.at[j]], urows.at[dst], sem))
        copies.append(pltpu.async_copy(ei_hbm.at[iidx.at[j]], irows.at[dst], sem))
    for c in copies:
        c.wait()

    lane = lax.iota(jnp.int32, 16)

    def body(g, carry):
        row = g * 16 + lane
        acc = jnp.zeros((16,), jnp.float32)
        for c in range(FACTOR):
            col = jnp.full((16,), c, jnp.int32)
            u = plsc.load_gather(urows, [row, col])
            v = plsc.load_gather(irows, [row, col])
            acc = acc + u * v
        out_v[pl.ds(g * 16, 16)] = acc
        return carry

    lax.fori_loop(0, GROUPS, body, 0)

    pltpu.sync_copy(out_v, out_hbm.at[pl.ds(base, B_PER_W)])


def kernel(user, item, embed_user, embed_item):
    user3 = user.astype(jnp.int32).reshape(NW, N_CHUNK, CHUNK)
    item3 = item.astype(jnp.int32).reshape(NW, N_CHUNK, CHUNK)
    return _pointmf_sc(user3, item3, embed_user, embed_item)


# speedup vs baseline: 1.0597x; 1.0597x over previous
import functools

import jax
import jax.numpy as jnp
from jax import lax
from jax.experimental import pallas as pl
from jax.experimental.pallas import tpu as pltpu
from jax.experimental.pallas import tpu_sc as plsc

BATCH = 16384
FACTOR = 64
WIDE = 128
NW = 32
B_PER_W = BATCH // NW
N_CHUNK = 4
CHUNK = B_PER_W // N_CHUNK
GROUPS = CHUNK // 16

_mesh = plsc.VectorSubcoreMesh(core_axis_name="c", subcore_axis_name="s")


@functools.partial(
    pl.kernel,
    mesh=_mesh,
    out_type=jax.ShapeDtypeStruct((BATCH,), jnp.float32),
    scratch_types=[
        pltpu.VMEM((N_CHUNK, CHUNK), jnp.int32),
        pltpu.VMEM((N_CHUNK, CHUNK), jnp.int32),
        pltpu.VMEM((CHUNK, WIDE), jnp.float32),
        pltpu.VMEM((CHUNK, WIDE), jnp.float32),
        pltpu.VMEM((B_PER_W,), jnp.float32),
        pltpu.SemaphoreType.DMA,
    ],
    compiler_params=pltpu.CompilerParams(
        needs_layout_passes=False, use_tc_tiling_on_sc=False
    ),
)
def _pointmf_sc(user_hbm, item_hbm, eu_hbm, ei_hbm, out_hbm,
                uidx, iidx, urows, irows, out_v, sem):
    wid = lax.axis_index("s") * 2 + lax.axis_index("c")
    base = wid * B_PER_W

    pltpu.sync_copy(user_hbm.at[wid], uidx)
    pltpu.sync_copy(item_hbm.at[wid], iidx)

    lane = lax.iota(jnp.int32, 16)

    for j in range(N_CHUNK):
        cu = pltpu.async_copy(eu_hbm.at[uidx.at[j]], urows, sem)
        ci = pltpu.async_copy(ei_hbm.at[iidx.at[j]], irows, sem)
        cu.wait()
        ci.wait()

        def body(g, carry):
            row = g * 16 + lane
            acc = jnp.zeros((16,), jnp.float32)
            for c in range(FACTOR):
                col = jnp.full((16,), c, jnp.int32)
                u = plsc.load_gather(urows, [row, col])
                v = plsc.load_gather(irows, [row, col])
                acc = acc + u * v
            out_v[pl.ds(j * CHUNK + g * 16, 16)] = acc
            return carry

        lax.fori_loop(0, GROUPS, body, 0)

    pltpu.sync_copy(out_v, out_hbm.at[pl.ds(base, B_PER_W)])


def kernel(user, item, embed_user, embed_item):
    user3 = user.astype(jnp.int32).reshape(NW, N_CHUNK, CHUNK)
    item3 = item.astype(jnp.int32).reshape(NW, N_CHUNK, CHUNK)
    eu = jnp.pad(embed_user, ((0, 0), (0, WIDE - FACTOR)))
    ei = jnp.pad(embed_item, ((0, 0), (0, WIDE - FACTOR)))
    return _pointmf_sc(user3, item3, eu, ei)
